# baseline (speedup 1.0000x reference)
import functools

import jax
import jax.numpy as jnp
from jax import lax
from jax.experimental import pallas as pl
from jax.experimental.pallas import tpu as pltpu
from jax.experimental.pallas import tpu_sc as plsc

N_NODES = 49152
F = 64
NVARS = 4
B = 2
V = 4

_NC = 2
_NS = 16
_NW = _NC * _NS

_R = NVARS * N_NODES
_RPW = _R // _NW
_IDX_PER_XFER = 128
_XFERS_PER_BLK = 8
_BLK_ROWS = _IDX_PER_XFER * _XFERS_PER_BLK
_NBLK = _RPW // _BLK_ROWS
_XPW = _RPW // _IDX_PER_XFER


def _sc_gather(table, idx3):
    mesh = plsc.VectorSubcoreMesh(core_axis_name="c", subcore_axis_name="s")

    @functools.partial(
        pl.kernel,
        out_type=jax.ShapeDtypeStruct((_R, F), jnp.float32),
        mesh=mesh,
        compiler_params=pltpu.CompilerParams(use_tc_tiling_on_sc=False),
        scratch_types=[
            pltpu.VMEM((_XPW, _IDX_PER_XFER), jnp.int32),
            pltpu.VMEM((_BLK_ROWS, F), jnp.float32),
            pltpu.SemaphoreType.DMA,
        ],
    )
    def gather_kernel(table_hbm, idx_hbm, out_hbm, idx_v, rows_v, sem):
        wid = lax.axis_index("s") * _NC + lax.axis_index("c")
        base = wid * _RPW
        pltpu.sync_copy(idx_hbm.at[wid], idx_v)

        def blk_body(blk, _):
            copies = []
            for j in range(_XFERS_PER_BLK):
                copies.append(pltpu.async_copy(
                    table_hbm.at[idx_v.at[blk * _XFERS_PER_BLK + j]],
                    rows_v.at[pl.ds(j * _IDX_PER_XFER, _IDX_PER_XFER)],
                    sem,
                ))
            for c in copies:
                c.wait()
            pltpu.sync_copy(
                rows_v,
                out_hbm.at[pl.ds(base + blk * _BLK_ROWS, _BLK_ROWS)],
            )
            return ()

        lax.fori_loop(0, _NBLK, blk_body, (), unroll=False)

    return gather_kernel(table, idx3)


_N2 = N_NODES // 2
_BN2 = 2048
_NB2 = _N2 // _BN2


def _film_body(vi_ref, x_ref, eg_ref, w_ref, b_ref, o_ref):
    bv = pl.program_id(1)
    vi = vi_ref[bv]
    eg = eg_ref[vi]
    m = jnp.dot(eg, w_ref[...], preferred_element_type=jnp.float32)
    m = m + b_ref[...]
    scale = m[:, : 2 * F]
    shift = m[:, 2 * F :]
    o_ref[0, 0, 0] = x_ref[0, 0, 0] * scale + shift


def _tc_film(vi, x2, eg2, W2, b2):
    grid = (_NB2, B * V)
    grid_spec = pltpu.PrefetchScalarGridSpec(
        num_scalar_prefetch=1,
        grid=grid,
        in_specs=[
            pl.BlockSpec(
                (1, 1, 1, _BN2, 2 * F),
                lambda n, bv, vi_ref: (bv // V, 0, bv % V, n, 0),
            ),
            pl.BlockSpec(
                (NVARS, _BN2, 2 * F),
                lambda n, bv, vi_ref: (0, n, 0),
            ),
            pl.BlockSpec((2 * F, 4 * F), lambda n, bv, vi_ref: (0, 0)),
            pl.BlockSpec((1, 4 * F), lambda n, bv, vi_ref: (0, 0)),
        ],
        out_specs=pl.BlockSpec(
            (1, 1, 1, _BN2, 2 * F),
            lambda n, bv, vi_ref: (bv // V, 0, bv % V, n, 0),
        ),
    )
    return pl.pallas_call(
        _film_body,
        grid_spec=grid_spec,
        out_shape=jax.ShapeDtypeStruct(x2.shape, x2.dtype),
    )(vi, x2, eg2, W2, b2)


def kernel(x, var_idx, adjc, embeddings, W, b):
    node_idx = adjc[:, 0].astype(jnp.int32)
    offs = (jnp.arange(NVARS, dtype=jnp.int32) * N_NODES)[:, None]
    idx3 = (offs + node_idx[None, :]).reshape(_NW, _XPW, _IDX_PER_XFER)
    table = embeddings.reshape(_R, F)
    eg2 = _sc_gather(table, idx3).reshape(NVARS, _N2, 2 * F)
    vi = var_idx.reshape(B * V).astype(jnp.int32)

    Ws, Wt = W[:, :F], W[:, F:]
    bs, bt = b[:F], b[F:]
    Z = jnp.zeros((F, F), dtype=W.dtype)
    W2 = jnp.concatenate(
        [
            jnp.concatenate([Ws, Z, Wt, Z], axis=1),
            jnp.concatenate([Z, Ws, Z, Wt], axis=1),
        ],
        axis=0,
    )
    b2 = jnp.concatenate([bs, bs, bt, bt]).reshape(1, 4 * F)

    x2 = x.reshape(B, 1, V, _N2, 2 * F)
    out2 = _tc_film(vi, x2, eg2, W2, b2)
    return out2.reshape(x.shape)


# speedup vs baseline: 1.1652x; 1.1652x over previous
import functools

import jax
import jax.numpy as jnp
from jax import lax
from jax.experimental import pallas as pl
from jax.experimental.pallas import tpu as pltpu
from jax.experimental.pallas import tpu_sc as plsc

N_NODES = 49152
F = 64
NVARS = 4
B = 2
V = 4

_NC = 2
_NS = 16
_NW = _NC * _NS

_R = NVARS * N_NODES
_RPW = _R // _NW
_NPW = _RPW
_IDX_PER_XFER = 128
_XFERS_PER_BLK = 8
_BLK_ROWS = _IDX_PER_XFER * _XFERS_PER_BLK
_NBLK = _RPW // _BLK_ROWS
_XPW = _RPW // _IDX_PER_XFER


def _sc_gather_paired(table, idx3):
    mesh = plsc.VectorSubcoreMesh(core_axis_name="c", subcore_axis_name="s")

    @functools.partial(
        pl.kernel,
        out_type=jax.ShapeDtypeStruct((NVARS // 2, N_NODES, 2 * F), jnp.float32),
        mesh=mesh,
        compiler_params=pltpu.CompilerParams(use_tc_tiling_on_sc=False),
        scratch_types=[
            pltpu.VMEM((_XPW, _IDX_PER_XFER), jnp.int32),
            pltpu.VMEM((_BLK_ROWS, F), jnp.float32),
            pltpu.SemaphoreType.DMA,
        ],
    )
    def gather_kernel(table_hbm, idx_hbm, out_hbm, idx_v, rows_v, sem):
        wid = lax.axis_index("s") * _NC + lax.axis_index("c")
        u = wid // 8
        node0 = (wid % 8) * _NPW
        p = u // 2
        q = u % 2
        pltpu.sync_copy(idx_hbm.at[wid], idx_v)

        def blk_body(blk, _):
            copies = []
            for j in range(_XFERS_PER_BLK):
                copies.append(pltpu.async_copy(
                    table_hbm.at[idx_v.at[blk * _XFERS_PER_BLK + j]],
                    rows_v.at[pl.ds(j * _IDX_PER_XFER, _IDX_PER_XFER)],
                    sem,
                ))
            for c in copies:
                c.wait()
            pltpu.sync_copy(
                rows_v,
                out_hbm.at[p, pl.ds(node0 + blk * _BLK_ROWS, _BLK_ROWS),
                           pl.ds(q * F, F)],
            )
            return ()

        lax.fori_loop(0, _NBLK, blk_body, (), unroll=False)

    return gather_kernel(table, idx3)


_BN = 4096
_NB = N_NODES // _BN


def _film_body(vi_ref, x_ref, eg_ref, w_ref, b_ref, o_ref):
    bv = pl.program_id(1)
    vi = vi_ref[bv]
    pair = eg_ref[vi // 2]
    m_all = jnp.dot(pair, w_ref[...], preferred_element_type=jnp.float32)
    m_all = m_all + b_ref[...]
    m = jnp.where(vi % 2 == 0, m_all[:, : 2 * F], m_all[:, 2 * F :])
    o_ref[0, 0, 0] = x_ref[0, 0, 0] * m[:, :F] + m[:, F:]


def _tc_film(vi, x, eg, W2, b2):
    grid = (_NB, B * V)
    grid_spec = pltpu.PrefetchScalarGridSpec(
        num_scalar_prefetch=1,
        grid=grid,
        in_specs=[
            pl.BlockSpec(
                (1, 1, 1, _BN, F),
                lambda n, bv, vi_ref: (bv // V, 0, bv % V, n, 0),
            ),
            pl.BlockSpec(
                (NVARS // 2, _BN, 2 * F),
                lambda n, bv, vi_ref: (0, n, 0),
            ),
            pl.BlockSpec((2 * F, 4 * F), lambda n, bv, vi_ref: (0, 0)),
            pl.BlockSpec((1, 4 * F), lambda n, bv, vi_ref: (0, 0)),
        ],
        out_specs=pl.BlockSpec(
            (1, 1, 1, _BN, F),
            lambda n, bv, vi_ref: (bv // V, 0, bv % V, n, 0),
        ),
    )
    return pl.pallas_call(
        _film_body,
        grid_spec=grid_spec,
        out_shape=jax.ShapeDtypeStruct(x.shape, x.dtype),
    )(vi, x, eg, W2, b2)


def kernel(x, var_idx, adjc, embeddings, W, b):
    node_idx = adjc[:, 0].astype(jnp.int32)
    offs = (jnp.arange(NVARS, dtype=jnp.int32) * N_NODES)[:, None]
    idx3 = (offs + node_idx[None, :]).reshape(_NW, _XPW, _IDX_PER_XFER)
    table = embeddings.reshape(_R, F)
    eg = _sc_gather_paired(table, idx3)
    vi = var_idx.reshape(B * V).astype(jnp.int32)

    Z = jnp.zeros((F, 2 * F), dtype=W.dtype)
    W2 = jnp.concatenate(
        [jnp.concatenate([W, Z], axis=1), jnp.concatenate([Z, W], axis=1)],
        axis=0,
    )
    b2 = jnp.concatenate([b, b]).reshape(1, 4 * F)

    return _tc_film(vi, x, eg, W2, b2)


# speedup vs baseline: 1.2251x; 1.0514x over previous
import functools

import jax
import jax.numpy as jnp
from jax import lax
from jax.experimental import pallas as pl
from jax.experimental.pallas import tpu as pltpu
from jax.experimental.pallas import tpu_sc as plsc

N_NODES = 49152
F = 64
NVARS = 4
B = 2
V = 4

_NC = 2
_NS = 16
_NW = _NC * _NS

_R = NVARS * N_NODES
_RPW = _R // _NW
_NPW = _RPW
_IDX_PER_XFER = 128
_XFERS_PER_BLK = 8
_BLK_ROWS = _IDX_PER_XFER * _XFERS_PER_BLK
_NBLK = _RPW // _BLK_ROWS
_XPW = _RPW // _IDX_PER_XFER


def _sc_gather_paired(table, idx3):
    mesh = plsc.VectorSubcoreMesh(core_axis_name="c", subcore_axis_name="s")

    @functools.partial(
        pl.kernel,
        out_type=jax.ShapeDtypeStruct((NVARS // 2, N_NODES, 2 * F), jnp.float32),
        mesh=mesh,
        compiler_params=pltpu.CompilerParams(use_tc_tiling_on_sc=False),
        scratch_types=[
            pltpu.VMEM((_XPW, _IDX_PER_XFER), jnp.int32),
            pltpu.VMEM((_BLK_ROWS, F), jnp.float32),
            pltpu.SemaphoreType.DMA,
        ],
    )
    def gather_kernel(table_hbm, idx_hbm, out_hbm, idx_v, rows_v, sem):
        wid = lax.axis_index("s") * _NC + lax.axis_index("c")
        u = wid // 8
        node0 = (wid % 8) * _NPW
        p = u // 2
        q = u % 2
        pltpu.sync_copy(idx_hbm.at[wid], idx_v)

        def blk_body(blk, _):
            copies = []
            for j in range(_XFERS_PER_BLK):
                copies.append(pltpu.async_copy(
                    table_hbm.at[idx_v.at[blk * _XFERS_PER_BLK + j]],
                    rows_v.at[pl.ds(j * _IDX_PER_XFER, _IDX_PER_XFER)],
                    sem,
                ))
            for c in copies:
                c.wait()
            pltpu.sync_copy(
                rows_v,
                out_hbm.at[p, pl.ds(node0 + blk * _BLK_ROWS, _BLK_ROWS),
                           pl.ds(q * F, F)],
            )
            return ()

        lax.fori_loop(0, _NBLK, blk_body, (), unroll=False)

    return gather_kernel(table, idx3)


_BN = 8192
_NB = N_NODES // _BN


def _film_body(vi_ref, x_ref, eg_ref, w_ref, b_ref, o_ref):
    bv = pl.program_id(1)
    vi = vi_ref[bv]
    pair = eg_ref[vi // 2]
    m_all = jnp.dot(pair, w_ref[...], preferred_element_type=jnp.float32)
    m_all = m_all + b_ref[...]
    m = jnp.where(vi % 2 == 0, m_all[:, : 2 * F], m_all[:, 2 * F :])
    o_ref[0, 0, 0] = x_ref[0, 0, 0] * m[:, :F] + m[:, F:]


def _tc_film(vi, x, eg, W2, b2):
    grid = (_NB, B * V)
    grid_spec = pltpu.PrefetchScalarGridSpec(
        num_scalar_prefetch=1,
        grid=grid,
        in_specs=[
            pl.BlockSpec(
                (1, 1, 1, _BN, F),
                lambda n, bv, vi_ref: (bv // V, 0, bv % V, n, 0),
            ),
            pl.BlockSpec(
                (NVARS // 2, _BN, 2 * F),
                lambda n, bv, vi_ref: (0, n, 0),
            ),
            pl.BlockSpec((2 * F, 4 * F), lambda n, bv, vi_ref: (0, 0)),
            pl.BlockSpec((1, 4 * F), lambda n, bv, vi_ref: (0, 0)),
        ],
        out_specs=pl.BlockSpec(
            (1, 1, 1, _BN, F),
            lambda n, bv, vi_ref: (bv // V, 0, bv % V, n, 0),
        ),
    )
    return pl.pallas_call(
        _film_body,
        grid_spec=grid_spec,
        out_shape=jax.ShapeDtypeStruct(x.shape, x.dtype),
    )(vi, x, eg, W2, b2)


def kernel(x, var_idx, adjc, embeddings, W, b):
    node_idx = adjc[:, 0].astype(jnp.int32)
    offs = (jnp.arange(NVARS, dtype=jnp.int32) * N_NODES)[:, None]
    idx3 = (offs + node_idx[None, :]).reshape(_NW, _XPW, _IDX_PER_XFER)
    table = embeddings.reshape(_R, F)
    eg = _sc_gather_paired(table, idx3)
    vi = var_idx.reshape(B * V).astype(jnp.int32)

    Z = jnp.zeros((F, 2 * F), dtype=W.dtype)
    W2 = jnp.concatenate(
        [jnp.concatenate([W, Z], axis=1), jnp.concatenate([Z, W], axis=1)],
        axis=0,
    )
    b2 = jnp.concatenate([b, b]).reshape(1, 4 * F)

    return _tc_film(vi, x, eg, W2, b2)
